# Initial kernel scaffold; baseline (speedup 1.0000x reference)
import functools

import jax
import jax.numpy as jnp
from jax import lax
from jax.experimental import pallas as pl
from jax.experimental.pallas import tpu as pltpu
from jax.experimental.pallas import tpu_sc as plsc


def kernel(src, table):
    B, H = src.shape
    V, D = table.shape
    N = B * H
    idx = src.reshape(N)

    info = plsc.get_sparse_core_info()
    NC, NS = info.num_cores, info.num_subcores
    NW = NC * NS
    n_per_w = N // NW
    C = 512
    n_chunks = n_per_w // C

    mesh = plsc.VectorSubcoreMesh(core_axis_name="c", subcore_axis_name="s")

    @functools.partial(
        pl.kernel,
        out_type=jax.ShapeDtypeStruct((N, D), jnp.float32),
        mesh=mesh,
        scratch_types=[
            pltpu.VMEM((C,), jnp.int32),
            pltpu.VMEM((C, D), jnp.float32),
            pltpu.SemaphoreType.DMA,
        ],
    )
    def gather_kernel(idx_hbm, table_hbm, out_hbm, idx_v, rows_v, sem):
        wid = lax.axis_index("s") * NC + lax.axis_index("c")
        base = wid * n_per_w

        def body(i, carry):
            off = base + i * C
            pltpu.sync_copy(idx_hbm.at[pl.ds(off, C)], idx_v)
            pltpu.async_copy(table_hbm.at[idx_v], rows_v, sem).wait()
            pltpu.sync_copy(rows_v, out_hbm.at[pl.ds(off, C)])
            return carry

        lax.fori_loop(0, n_chunks, body, 0)

    out = gather_kernel(idx, table)
    return out.reshape(B, H, D)


# speedup vs baseline: 3.5401x; 3.5401x over previous
import functools

import jax
import jax.numpy as jnp
from jax import lax
from jax.experimental import pallas as pl
from jax.experimental.pallas import tpu as pltpu
from jax.experimental.pallas import tpu_sc as plsc


def kernel(src, table):
    B, H = src.shape
    V, D = table.shape
    N = B * H
    idx = src.reshape(N)

    info = plsc.get_sparse_core_info()
    NC, NS = info.num_cores, info.num_subcores
    NW = NC * NS
    n_per_w = N // NW
    C = 512
    n_chunks = n_per_w // C

    mesh = plsc.VectorSubcoreMesh(core_axis_name="c", subcore_axis_name="s")

    @functools.partial(
        pl.kernel,
        out_type=jax.ShapeDtypeStruct((N, D), jnp.float32),
        mesh=mesh,
        scratch_types=[
            pltpu.VMEM((C,), jnp.int32),
            pltpu.VMEM((C, D), jnp.float32),
            pltpu.SemaphoreType.DMA,
        ],
        compiler_params=pltpu.CompilerParams(use_tc_tiling_on_sc=False),
    )
    def gather_kernel(idx_hbm, table_hbm, out_hbm, idx_v, rows_v, sem):
        wid = lax.axis_index("s") * NC + lax.axis_index("c")
        base = wid * n_per_w

        def body(i, carry):
            off = base + i * C
            pltpu.sync_copy(idx_hbm.at[pl.ds(off, C)], idx_v)
            pltpu.async_copy(table_hbm.at[idx_v], rows_v, sem).wait()
            pltpu.sync_copy(rows_v, out_hbm.at[pl.ds(off, C)])
            return carry

        lax.fori_loop(0, n_chunks, body, 0)

    out = gather_kernel(idx, table)
    return out.reshape(B, H, D)


# speedup vs baseline: 3.5608x; 1.0058x over previous
import functools

import jax
import jax.numpy as jnp
from jax import lax
from jax.experimental import pallas as pl
from jax.experimental.pallas import tpu as pltpu
from jax.experimental.pallas import tpu_sc as plsc


def kernel(src, table):
    B, H = src.shape
    V, D = table.shape
    N = B * H
    idx = src.reshape(N)

    info = plsc.get_sparse_core_info()
    NC, NS = info.num_cores, info.num_subcores
    NW = NC * NS
    n_per_w = N // NW
    C = 800
    n_chunks = n_per_w // C
    assert n_chunks * C == n_per_w and n_chunks % 2 == 0 and n_chunks >= 4

    mesh = plsc.VectorSubcoreMesh(core_axis_name="c", subcore_axis_name="s")

    @functools.partial(
        pl.kernel,
        out_type=jax.ShapeDtypeStruct((N, D), jnp.float32),
        mesh=mesh,
        scratch_types=[
            pltpu.VMEM((C,), jnp.int32),
            pltpu.VMEM((C,), jnp.int32),
            pltpu.VMEM((C, D), jnp.float32),
            pltpu.VMEM((C, D), jnp.float32),
            pltpu.SemaphoreType.DMA,
            pltpu.SemaphoreType.DMA,
            pltpu.SemaphoreType.DMA,
            pltpu.SemaphoreType.DMA,
        ],
        compiler_params=pltpu.CompilerParams(use_tc_tiling_on_sc=False),
    )
    def gather_kernel(idx_hbm, table_hbm, out_hbm,
                      idx0, idx1, rows0, rows1, sg0, sg1, ss0, ss1):
        wid = lax.axis_index("s") * NC + lax.axis_index("c")
        base = wid * n_per_w
        idx_v = (idx0, idx1)
        rows_v = (rows0, rows1)
        sg = (sg0, sg1)
        ss = (ss0, ss1)

        def idx_copy(i, b):
            pltpu.sync_copy(idx_hbm.at[pl.ds(base + i * C, C)], idx_v[b])

        def gather_start(b):
            pltpu.async_copy(table_hbm.at[idx_v[b]], rows_v[b], sg[b])

        def gather_wait(b):
            pltpu.make_async_copy(table_hbm.at[idx_v[b]], rows_v[b], sg[b]).wait()

        def scatter_start(i, b):
            pltpu.async_copy(rows_v[b], out_hbm.at[pl.ds(base + i * C, C)], ss[b])

        def scatter_wait(i, b):
            pltpu.make_async_copy(
                rows_v[b], out_hbm.at[pl.ds(base + i * C, C)], ss[b]).wait()

        idx_copy(0, 0)
        gather_start(0)
        idx_copy(1, 1)
        gather_start(1)

        gather_wait(0)
        scatter_start(0, 0)

        def body(g, carry):
            for t in (0, 1):
                i = 1 + 2 * g + t
                b = (1 + t) % 2
                gather_wait(b)
                scatter_start(i, b)
                scatter_wait(i - 1, 1 - b)
                idx_copy(i + 1, 1 - b)
                gather_start(1 - b)
            return carry

        lax.fori_loop(0, (n_chunks - 2) // 2, body, 0)

        i = n_chunks - 1
        gather_wait(1)
        scatter_start(i, 1)
        scatter_wait(i - 1, 0)
        scatter_wait(i, 1)

    out = gather_kernel(idx, table)
    return out.reshape(B, H, D)
